# Initial kernel scaffold; baseline (speedup 1.0000x reference)
import jax
import jax.numpy as jnp
from jax.experimental import pallas as pl


def kernel(features, labels, prototypes):
    raise NotImplementedError("write your pallas kernel here")


# speedup vs baseline: 3.4439x; 3.4439x over previous
import functools

import jax
import jax.numpy as jnp
from jax import lax
from jax.experimental import pallas as pl
from jax.experimental.pallas import tpu as pltpu
from jax.experimental.pallas import tpu_sc as plsc

NUM_CLASSES = 1000
FEAT_DIM = 128
BATCH = 16384
ALPHA = 0.99

PAD_CLASSES = 1024
NW = 16
ROWS_PER_W = BATCH // NW
CHUNK = 128
NCHUNK = ROWS_PER_W // CHUNK
CLS_PER_W = PAD_CLASSES // NW
LANES = 16


def _body(feat_hbm, lbl_hbm, proto_hbm, out_hbm,
          lbl_v, feat_v, ones_v, acc_v, cnt_v, proto_v, out_v,
          shared_acc, shared_cnt):
    wid = lax.axis_index("s")
    cls_base = wid * CLS_PER_W

    zeros16 = jnp.zeros((LANES,), jnp.float32)

    def zero_row(r, _):
        for j in range(FEAT_DIM // LANES):
            acc_v[r, pl.ds(j * LANES, LANES)] = zeros16
        return _

    lax.fori_loop(0, CLS_PER_W, zero_row, None)

    ones16 = jnp.ones((LANES,), jnp.float32)

    def ones_row(r, _):
        for j in range(FEAT_DIM // LANES):
            ones_v[r, pl.ds(j * LANES, LANES)] = ones16
        return _

    lax.fori_loop(0, CHUNK, ones_row, None)

    pltpu.sync_copy(acc_v, shared_acc.at[pl.ds(cls_base, CLS_PER_W)])
    pltpu.sync_copy(acc_v, shared_cnt.at[pl.ds(cls_base, CLS_PER_W)])
    plsc.subcore_barrier()

    pltpu.sync_copy(lbl_hbm.at[wid], lbl_v)
    for j in range(NCHUNK):
        row_base = wid * ROWS_PER_W + j * CHUNK
        pltpu.sync_copy(feat_hbm.at[pl.ds(row_base, CHUNK)], feat_v)
        pltpu.sync_copy(feat_v, shared_acc.at[lbl_v.at[j]], add=True)
        pltpu.sync_copy(ones_v, shared_cnt.at[lbl_v.at[j]], add=True)
    plsc.subcore_barrier()

    pltpu.sync_copy(shared_acc.at[pl.ds(cls_base, CLS_PER_W)], acc_v)
    pltpu.sync_copy(shared_cnt.at[pl.ds(cls_base, CLS_PER_W)], cnt_v)
    pltpu.sync_copy(proto_hbm.at[pl.ds(cls_base, CLS_PER_W)], proto_v)

    def ema_row(c, _):
        cnt = cnt_v[c, pl.ds(0, LANES)]
        present = cnt > 0.0
        inv = 1.0 / jnp.maximum(cnt, 1.0)
        for j in range(FEAT_DIM // LANES):
            s = acc_v[c, pl.ds(j * LANES, LANES)]
            p = proto_v[c, pl.ds(j * LANES, LANES)]
            out_v[c, pl.ds(j * LANES, LANES)] = jnp.where(
                present, ALPHA * p + (1.0 - ALPHA) * (s * inv), p)
        return _

    lax.fori_loop(0, CLS_PER_W, ema_row, None)
    pltpu.sync_copy(out_v, out_hbm.at[pl.ds(cls_base, CLS_PER_W)])


@jax.jit
def _run(features, labels3, proto_pad):
    mesh = plsc.VectorSubcoreMesh(
        core_axis_name="c", subcore_axis_name="s", num_cores=1,
        num_subcores=NW)
    call = pl.kernel(
        _body,
        out_type=jax.ShapeDtypeStruct((PAD_CLASSES, FEAT_DIM), jnp.float32),
        mesh=mesh,
        scratch_types=[
            pltpu.VMEM((NCHUNK, CHUNK), jnp.int32),
            pltpu.VMEM((CHUNK, FEAT_DIM), jnp.float32),
            pltpu.VMEM((CHUNK, FEAT_DIM), jnp.float32),
            pltpu.VMEM((CLS_PER_W, FEAT_DIM), jnp.float32),
            pltpu.VMEM((CLS_PER_W, FEAT_DIM), jnp.float32),
            pltpu.VMEM((CLS_PER_W, FEAT_DIM), jnp.float32),
            pltpu.VMEM((CLS_PER_W, FEAT_DIM), jnp.float32),
            pltpu.VMEM_SHARED((PAD_CLASSES, FEAT_DIM), jnp.float32),
            pltpu.VMEM_SHARED((PAD_CLASSES, FEAT_DIM), jnp.float32),
        ],
    )
    return call(features, labels3, proto_pad)


def kernel(features, labels, prototypes):
    labels3 = labels.astype(jnp.int32).reshape(NW, NCHUNK, CHUNK)
    proto_pad = jnp.pad(prototypes,
                        ((0, PAD_CLASSES - NUM_CLASSES), (0, 0)))
    out = _run(features, labels3, proto_pad)
    return out[:NUM_CLASSES]


# speedup vs baseline: 4.0678x; 1.1812x over previous
import jax
import jax.numpy as jnp
from jax import lax
from jax.experimental import pallas as pl
from jax.experimental.pallas import tpu as pltpu
from jax.experimental.pallas import tpu_sc as plsc

NUM_CLASSES = 1000
FEAT_DIM = 128
BATCH = 16384
ALPHA = 0.99

PAD_CLASSES = 1024
NW = 16
ROWS_PER_W = BATCH // NW
CHUNK = 128
NCHUNK = ROWS_PER_W // CHUNK
CLS_PER_W = PAD_CLASSES // NW
LANES = 16
VL = FEAT_DIM // LANES


def _body(feat_hbm, lbl_hbm, proto_hbm, out_hbm,
          lbl_v, feat_v, ones_v, acc_v, cnt_v, proto_v, out_v,
          shared_acc, shared_cnt, sem0, sem1):
    wid = lax.axis_index("s")
    cls_base = wid * CLS_PER_W
    zeros16 = jnp.zeros((LANES,), jnp.float32)
    ones16 = jnp.ones((LANES,), jnp.float32)

    def zero_row(r, _):
        for j in range(VL):
            acc_v[r, pl.ds(j * LANES, LANES)] = zeros16
        return _
    lax.fori_loop(0, CLS_PER_W, zero_row, None)

    pltpu.sync_copy(acc_v, shared_acc.at[pl.ds(cls_base, CLS_PER_W)])
    pltpu.sync_copy(acc_v, shared_cnt.at[pl.ds(cls_base, CLS_PER_W)])
    plsc.subcore_barrier()

    pltpu.sync_copy(lbl_hbm.at[wid], lbl_v)
    sems = [sem0, sem1]
    copies = [None, None]
    copies[0] = pltpu.async_copy(
        feat_hbm.at[pl.ds(wid * ROWS_PER_W, CHUNK)], feat_v.at[0], sem0)

    def ones_row(r, _):
        for j in range(VL):
            ones_v[r, pl.ds(j * LANES, LANES)] = ones16
        return _
    lax.fori_loop(0, CHUNK, ones_row, None)

    for j in range(NCHUNK):
        b = j % 2
        copies[b].wait()
        if j + 1 < NCHUNK:
            nb = (j + 1) % 2
            copies[nb] = pltpu.async_copy(
                feat_hbm.at[pl.ds(wid * ROWS_PER_W + (j + 1) * CHUNK, CHUNK)],
                feat_v.at[nb], sems[nb])
        pltpu.sync_copy(feat_v.at[b], shared_acc.at[lbl_v.at[j]], add=True)
        pltpu.sync_copy(ones_v, shared_cnt.at[lbl_v.at[j]], add=True)
    plsc.subcore_barrier()

    pltpu.sync_copy(shared_acc.at[pl.ds(cls_base, CLS_PER_W)], acc_v)
    pltpu.sync_copy(shared_cnt.at[pl.ds(cls_base, CLS_PER_W)], cnt_v)
    pltpu.sync_copy(proto_hbm.at[pl.ds(cls_base, CLS_PER_W)], proto_v)

    def ema_row(c, _):
        cnt = cnt_v[c, pl.ds(0, LANES)]
        present = cnt > 0.0
        inv = 1.0 / jnp.maximum(cnt, 1.0)
        for j in range(VL):
            s = acc_v[c, pl.ds(j * LANES, LANES)]
            p = proto_v[c, pl.ds(j * LANES, LANES)]
            out_v[c, pl.ds(j * LANES, LANES)] = jnp.where(
                present, ALPHA * p + (1.0 - ALPHA) * (s * inv), p)
        return _
    lax.fori_loop(0, CLS_PER_W, ema_row, None)
    pltpu.sync_copy(out_v, out_hbm.at[pl.ds(cls_base, CLS_PER_W)])


@jax.jit
def _run(features, labels3, proto_pad):
    mesh = plsc.VectorSubcoreMesh(
        core_axis_name="c", subcore_axis_name="s", num_cores=1,
        num_subcores=NW)
    call = pl.kernel(
        _body,
        out_type=jax.ShapeDtypeStruct((PAD_CLASSES, FEAT_DIM), jnp.float32),
        mesh=mesh,
        scratch_types=[
            pltpu.VMEM((NCHUNK, CHUNK), jnp.int32),
            pltpu.VMEM((2, CHUNK, FEAT_DIM), jnp.float32),
            pltpu.VMEM((CHUNK, FEAT_DIM), jnp.float32),
            pltpu.VMEM((CLS_PER_W, FEAT_DIM), jnp.float32),
            pltpu.VMEM((CLS_PER_W, FEAT_DIM), jnp.float32),
            pltpu.VMEM((CLS_PER_W, FEAT_DIM), jnp.float32),
            pltpu.VMEM((CLS_PER_W, FEAT_DIM), jnp.float32),
            pltpu.VMEM_SHARED((PAD_CLASSES, FEAT_DIM), jnp.float32),
            pltpu.VMEM_SHARED((PAD_CLASSES, FEAT_DIM), jnp.float32),
            pltpu.SemaphoreType.DMA,
            pltpu.SemaphoreType.DMA,
        ],
    )
    return call(features, labels3, proto_pad)


def kernel(features, labels, prototypes):
    labels3 = labels.astype(jnp.int32).reshape(NW, NCHUNK, CHUNK)
    proto_pad = jnp.pad(prototypes,
                        ((0, PAD_CLASSES - NUM_CLASSES), (0, 0)))
    out = _run(features, labels3, proto_pad)
    return out[:NUM_CLASSES]


# speedup vs baseline: 4.8199x; 1.1849x over previous
import jax
import jax.numpy as jnp
from jax import lax
from jax.experimental import pallas as pl
from jax.experimental.pallas import tpu as pltpu
from jax.experimental.pallas import tpu_sc as plsc

NUM_CLASSES = 1000
FEAT_DIM = 128
BATCH = 16384
ALPHA = 0.99

PAD_CLASSES = 1024
NC = 2
NS = 16
NWT = NC * NS
ROWS_PER_W = BATCH // NWT
CHUNK = 128
NCHUNK = ROWS_PER_W // CHUNK
CLS_PER_S = PAD_CLASSES // NS
LANES = 16
VL = FEAT_DIM // LANES


def _scatter_body(feat_hbm, lbl_hbm, psum_hbm, pcnt_hbm,
                  lbl_v, feat_v, ones_v, zero_v,
                  shared_acc, shared_cnt, sem0, sem1):
    cid = lax.axis_index("c")
    sid = lax.axis_index("s")
    wid = cid * NS + sid
    cls_base = sid * CLS_PER_S
    zeros16 = jnp.zeros((LANES,), jnp.float32)
    ones16 = jnp.ones((LANES,), jnp.float32)

    def zero_row(r, _):
        for j in range(VL):
            zero_v[r, pl.ds(j * LANES, LANES)] = zeros16
        return _
    lax.fori_loop(0, CLS_PER_S, zero_row, None)

    pltpu.sync_copy(zero_v, shared_acc.at[pl.ds(cls_base, CLS_PER_S)])
    pltpu.sync_copy(zero_v, shared_cnt.at[pl.ds(cls_base, CLS_PER_S)])
    plsc.subcore_barrier()

    pltpu.sync_copy(lbl_hbm.at[wid], lbl_v)
    sems = [sem0, sem1]
    copies = [None, None]
    copies[0] = pltpu.async_copy(
        feat_hbm.at[pl.ds(wid * ROWS_PER_W, CHUNK)], feat_v.at[0], sem0)

    def ones_row(r, _):
        for j in range(VL):
            ones_v[r, pl.ds(j * LANES, LANES)] = ones16
        return _
    lax.fori_loop(0, CHUNK, ones_row, None)

    for j in range(NCHUNK):
        b = j % 2
        copies[b].wait()
        if j + 1 < NCHUNK:
            nb = (j + 1) % 2
            copies[nb] = pltpu.async_copy(
                feat_hbm.at[pl.ds(wid * ROWS_PER_W + (j + 1) * CHUNK, CHUNK)],
                feat_v.at[nb], sems[nb])
        pltpu.sync_copy(feat_v.at[b], shared_acc.at[lbl_v.at[j]], add=True)
        pltpu.sync_copy(ones_v, shared_cnt.at[lbl_v.at[j]], add=True)
    plsc.subcore_barrier()

    pltpu.sync_copy(shared_acc.at[pl.ds(cls_base, CLS_PER_S)],
                    psum_hbm.at[cid, pl.ds(cls_base, CLS_PER_S)])
    pltpu.sync_copy(shared_cnt.at[pl.ds(cls_base, CLS_PER_S)],
                    pcnt_hbm.at[cid, pl.ds(cls_base, CLS_PER_S)])


def _ema_body(psum_ref, pcnt_ref, proto_ref, out_ref):
    s = psum_ref[0, :NUM_CLASSES, :] + psum_ref[1, :NUM_CLASSES, :]
    c = pcnt_ref[0, :NUM_CLASSES, :] + pcnt_ref[1, :NUM_CLASSES, :]
    p = proto_ref[...]
    mean = s / jnp.maximum(c, 1.0)
    out_ref[...] = jnp.where(c > 0.0, ALPHA * p + (1.0 - ALPHA) * mean, p)


@jax.jit
def _run(features, labels3, prototypes):
    mesh = plsc.VectorSubcoreMesh(
        core_axis_name="c", subcore_axis_name="s", num_cores=NC,
        num_subcores=NS)
    psum, pcnt = pl.kernel(
        _scatter_body,
        out_type=(
            jax.ShapeDtypeStruct((NC, PAD_CLASSES, FEAT_DIM), jnp.float32),
            jax.ShapeDtypeStruct((NC, PAD_CLASSES, FEAT_DIM), jnp.float32)),
        mesh=mesh,
        scratch_types=[
            pltpu.VMEM((NCHUNK, CHUNK), jnp.int32),
            pltpu.VMEM((2, CHUNK, FEAT_DIM), jnp.float32),
            pltpu.VMEM((CHUNK, FEAT_DIM), jnp.float32),
            pltpu.VMEM((CLS_PER_S, FEAT_DIM), jnp.float32),
            pltpu.VMEM_SHARED((PAD_CLASSES, FEAT_DIM), jnp.float32),
            pltpu.VMEM_SHARED((PAD_CLASSES, FEAT_DIM), jnp.float32),
            pltpu.SemaphoreType.DMA,
            pltpu.SemaphoreType.DMA,
        ],
    )(features, labels3)

    out = pl.pallas_call(
        _ema_body,
        out_shape=jax.ShapeDtypeStruct((NUM_CLASSES, FEAT_DIM), jnp.float32),
    )(psum, pcnt, prototypes)
    return out


def kernel(features, labels, prototypes):
    labels3 = labels.astype(jnp.int32).reshape(NWT, NCHUNK, CHUNK)
    return _run(features, labels3, prototypes)
